# baseline (speedup 1.0000x reference)
import functools

import jax
import jax.numpy as jnp
from jax import lax
from jax.experimental import pallas as pl
from jax.experimental.pallas import tpu as pltpu
from jax.experimental.pallas import tpu_sc as plsc

N = 10000
E = 320000
F = 128
FW = F // 2
NC = 2
NS = 16
NW = NC * NS
ERW = E // NW
CH = 128
NB = 2
EPW = 10240
NCH = EPW // CH
HCH = NCH // 2
NG = HCH // NB
L = 16
NPAD = 10240
NPS = NPAD // NS
ZR = 16

_MESH = dict(core_axis_name="c", subcore_axis_name="s", num_cores=NC,
             num_subcores=NS)


def _zero_1d(ref, n):
    def body(i, _):
        ref[pl.ds(i * L, L)] = jnp.zeros((L,), jnp.float32)
        return 0
    lax.fori_loop(0, n // L, body, 0)


def _rsum_body(rows_hbm, vals_hbm, out_hbm, rows_v, vals_v, acc_v, part_v,
               red_v, shared):
    c = lax.axis_index("c")
    s = lax.axis_index("s")
    wid = c * NS + s
    _zero_1d(acc_v, NPAD)
    pltpu.sync_copy(rows_hbm.at[pl.ds(wid * EPW, EPW)], rows_v)
    pltpu.sync_copy(vals_hbm.at[pl.ds(wid * EPW, EPW)], vals_v)

    def body(i, _):
        idx = rows_v[pl.ds(i * L, L)]
        v = vals_v[pl.ds(i * L, L)]
        plsc.addupdate_scatter(acc_v, [idx], v)
        return 0
    lax.fori_loop(0, EPW // L, body, 0)

    pltpu.sync_copy(acc_v, shared.at[s])
    plsc.subcore_barrier()
    _zero_1d(red_v, NPS)
    for t in range(NS):
        pltpu.sync_copy(shared.at[t, pl.ds(s * NPS, NPS)], part_v)

        def addb(i, _):
            red_v[pl.ds(i * L, L)] = (red_v[pl.ds(i * L, L)]
                                      + part_v[pl.ds(i * L, L)])
            return 0
        lax.fori_loop(0, NPS // L, addb, 0)
    pltpu.sync_copy(red_v, out_hbm.at[c, pl.ds(s * NPS, NPS)])


_rsum_call = functools.partial(
    pl.kernel,
    out_type=jax.ShapeDtypeStruct((NC, NPAD), jnp.float32),
    mesh=plsc.VectorSubcoreMesh(**_MESH),
    compiler_params=pltpu.CompilerParams(needs_layout_passes=False),
    scratch_types=[
        pltpu.VMEM((EPW,), jnp.int32),
        pltpu.VMEM((EPW,), jnp.float32),
        pltpu.VMEM((NPAD,), jnp.float32),
        pltpu.VMEM((NPS,), jnp.float32),
        pltpu.VMEM((NPS,), jnp.float32),
        pltpu.VMEM_SHARED((NS, NPAD), jnp.float32),
    ],
)(_rsum_body)


def _we_body(rs_hbm, rows_hbm, cols_hbm, vals_hbm, we_hbm, rc_hbm, rs0_v,
             rs1_v, d_v, rows_v, cols_v, vals_v, we_v, rc_v):
    c = lax.axis_index("c")
    s = lax.axis_index("s")
    wid = c * NS + s
    pltpu.sync_copy(rs_hbm.at[0], rs0_v)
    pltpu.sync_copy(rs_hbm.at[1], rs1_v)

    def dbody(i, _):
        x = rs0_v[pl.ds(i * L, L)] + rs1_v[pl.ds(i * L, L)] + 1.0
        xi = plsc.bitcast(x, jnp.int32)
        yi = 0x5F3759DF - lax.shift_right_arithmetic(xi, 1)
        y = plsc.bitcast(yi, jnp.float32)
        hx = 0.5 * x
        for _ in range(4):
            y = y * (1.5 - hx * y * y)
        d_v[pl.ds(i * L, L)] = y
        return 0
    lax.fori_loop(0, NPAD // L, dbody, 0)

    pltpu.sync_copy(rows_hbm.at[pl.ds(wid * EPW, EPW)], rows_v)
    pltpu.sync_copy(cols_hbm.at[pl.ds(wid * EPW, EPW)], cols_v)
    pltpu.sync_copy(vals_hbm.at[pl.ds(wid * EPW, EPW)], vals_v)

    def ebody(i, _):
        r = rows_v[pl.ds(i * L, L)]
        cc = cols_v[pl.ds(i * L, L)]
        dr = plsc.load_gather(d_v, [r])
        dc = plsc.load_gather(d_v, [cc])
        we_v[pl.ds(i * L, L)] = vals_v[pl.ds(i * L, L)] * dr * dc
        rc_v[pl.ds(i * L, L)] = lax.bitwise_or(
            r, lax.shift_left(cc, jnp.int32(16)))
        return 0
    lax.fori_loop(0, EPW // L, ebody, 0)
    pltpu.sync_copy(we_v, we_hbm.at[pl.ds(wid * EPW, EPW)])
    pltpu.sync_copy(rc_v, rc_hbm.at[pl.ds(wid * EPW, EPW)])


_we_call = functools.partial(
    pl.kernel,
    out_type=[jax.ShapeDtypeStruct((NW * EPW,), jnp.float32),
              jax.ShapeDtypeStruct((NW * EPW,), jnp.int32)],
    mesh=plsc.VectorSubcoreMesh(**_MESH),
    compiler_params=pltpu.CompilerParams(needs_layout_passes=False),
    scratch_types=[
        pltpu.VMEM((NPAD,), jnp.float32),
        pltpu.VMEM((NPAD,), jnp.float32),
        pltpu.VMEM((NPAD,), jnp.float32),
        pltpu.VMEM((EPW,), jnp.int32),
        pltpu.VMEM((EPW,), jnp.int32),
        pltpu.VMEM((EPW,), jnp.float32),
        pltpu.VMEM((EPW,), jnp.float32),
        pltpu.VMEM((EPW,), jnp.int32),
    ],
)(_we_body)


def _hop_body(x_hbm, rc3_hbm, we3_hbm, out_hbm, rc2_v, we2_v, cb0, cb1,
              rowb_v, gb0, gb1, sbuf_v, zbuf_v, acc_sh, sg0, sg1, ssem):
    c = lax.axis_index("c")
    s = lax.axis_index("s")
    wid = c * NS + s
    gbufs = (gb0, gb1)
    colbs = (cb0, cb1)
    gsems = (sg0, sg1)

    def zrow(r, _):
        for g in range(F // L):
            zbuf_v[r, pl.ds(g * L, L)] = jnp.zeros((L,), jnp.float32)
        return 0
    lax.fori_loop(0, ZR, zrow, 0)
    for b in range(NPS // ZR):
        pltpu.sync_copy(zbuf_v, acc_sh.at[pl.ds(s * NPS + b * ZR, ZR)])
    plsc.subcore_barrier()

    def unpack_cols(j, cb):
        def ub(q, _):
            v = rc2_v[j, pl.ds(q * L, L)]
            cb[pl.ds(q * L, L)] = lax.shift_right_logical(v, jnp.int32(16))
            return 0
        lax.fori_loop(0, CH // L, ub, 0)

    def unpack_rows(j):
        def ub(q, _):
            v = rc2_v[j, pl.ds(q * L, L)]
            rowb_v[pl.ds(q * L, L)] = lax.bitwise_and(v, jnp.int32(0xFFFF))
            return 0
        lax.fori_loop(0, CH // L, ub, 0)

    def mult(src, j):
        def qbody(q, _):
            wv = we2_v[j, pl.ds(q * L, L)]
            for lane in range(L):
                w = wv[lane]
                e = q * L + lane
                for g in range(F // 32):
                    v = src[e, pl.ds(g * L, L)]
                    lo = plsc.bitcast(
                        lax.shift_left(v, jnp.int32(16)), jnp.float32)
                    hi = plsc.bitcast(
                        lax.bitwise_and(v, jnp.int32(-65536)), jnp.float32)
                    sbuf_v[e, pl.ds(g * 32, L)] = lo * w
                    sbuf_v[e, pl.ds(g * 32 + L, L)] = hi * w
            return 0
        lax.fori_loop(0, CH // L, qbody, 0)

    def chunk(j, b, first):
        pltpu.make_async_copy(x_hbm.at[colbs[b]], gbufs[b], gsems[b]).wait()
        if not first:
            pltpu.make_async_copy(sbuf_v, acc_sh.at[rowb_v], ssem).wait()
        unpack_rows(j)
        mult(gbufs[b], j)
        pltpu.async_copy(sbuf_v, acc_sh.at[rowb_v], ssem, add=True)
        unpack_cols(j + NB, colbs[b])
        pltpu.async_copy(x_hbm.at[colbs[b]], gbufs[b], gsems[b])

    for p in range(2):
        pltpu.sync_copy(rc3_hbm.at[wid, pl.ds(p * HCH, HCH + NB)], rc2_v)
        pltpu.sync_copy(we3_hbm.at[wid, pl.ds(p * HCH, HCH)], we2_v)
        for b in range(NB):
            unpack_cols(b, colbs[b])
            pltpu.async_copy(x_hbm.at[colbs[b]], gbufs[b], gsems[b])
        chunk(0, 0, True)
        chunk(1, 1, False)

        def group(g, _):
            for b in range(NB):
                chunk(g * NB + b, b, False)
            return 0
        lax.fori_loop(1, NG, group, 0)

        for b in range(NB):
            pltpu.make_async_copy(x_hbm.at[colbs[b]], gbufs[b],
                                  gsems[b]).wait()
        pltpu.make_async_copy(sbuf_v, acc_sh.at[rowb_v], ssem).wait()

    plsc.subcore_barrier()
    pltpu.sync_copy(acc_sh.at[pl.ds(s * NPS, NPS)],
                    out_hbm.at[c, pl.ds(s * NPS, NPS)])


_hop_call = functools.partial(
    pl.kernel,
    out_type=jax.ShapeDtypeStruct((NC, NPAD, F), jnp.float32),
    mesh=plsc.VectorSubcoreMesh(**_MESH),
    compiler_params=pltpu.CompilerParams(needs_layout_passes=False,
                                         use_tc_tiling_on_sc=False),
    scratch_types=[
        pltpu.VMEM((HCH + NB, CH), jnp.int32),
        pltpu.VMEM((HCH, CH), jnp.float32),
        pltpu.VMEM((CH,), jnp.int32),
        pltpu.VMEM((CH,), jnp.int32),
        pltpu.VMEM((CH,), jnp.int32),
        pltpu.VMEM((CH, FW), jnp.int32),
        pltpu.VMEM((CH, FW), jnp.int32),
        pltpu.VMEM((CH, F), jnp.float32),
        pltpu.VMEM((ZR, F), jnp.float32),
        pltpu.VMEM_SHARED((NPAD, F), jnp.float32),
        pltpu.SemaphoreType.DMA,
        pltpu.SemaphoreType.DMA,
        pltpu.SemaphoreType.DMA,
    ],
)(_hop_body)


_RB = 1000


def _pack_bf16(x):
    u = lax.bitcast_convert_type(x.astype(jnp.bfloat16), jnp.uint16)
    u = u.astype(jnp.uint32)
    parts = []
    for g in range(F // 32):
        lo = u[:, g * 32:g * 32 + 16]
        hi = u[:, g * 32 + 16:g * 32 + 32]
        parts.append(lax.bitwise_or(lax.shift_left(hi, jnp.uint32(16)), lo))
    return lax.bitcast_convert_type(jnp.concatenate(parts, axis=1),
                                    jnp.int32)


def _mk_tc(first, last):
    def body(*refs):
        a0, a1, w = refs[:3]
        rest = list(refs[3:])
        h_prev = None if first else rest.pop(0)[...]
        b = rest.pop(0)[...] if last else None
        xp_o, h_o = rest
        xa = a0[...] + a1[...]
        xp_o[...] = _pack_bf16(xa)
        acc = jnp.dot(xa, w[...], preferred_element_type=jnp.float32)
        if h_prev is not None:
            acc = acc + h_prev
        if b is not None:
            acc = acc + b
        h_o[...] = acc

    row_spec = pl.BlockSpec((_RB, F), lambda i: (i, 0))
    pack_spec = pl.BlockSpec((_RB, FW), lambda i: (i, 0))
    w_spec = pl.BlockSpec((F, F), lambda i: (0, 0))
    bias_spec = pl.BlockSpec((1, F), lambda i: (0, 0))
    in_specs = [row_spec, row_spec, w_spec]
    if not first:
        in_specs.append(row_spec)
    if last:
        in_specs.append(bias_spec)
    return pl.pallas_call(
        body,
        grid=(N // _RB,),
        in_specs=in_specs,
        out_specs=[pack_spec, row_spec],
        out_shape=[jax.ShapeDtypeStruct((N, FW), jnp.int32),
                   jax.ShapeDtypeStruct((N, F), jnp.float32)],
    )


_tc_first = _mk_tc(True, False)
_tc_mid = _mk_tc(False, False)
_tc_last = _mk_tc(False, True)


def kernel(edge_index, edge_vals, X, weights, bias):
    pad = ((0, 0), (0, EPW - ERW))
    rows = jnp.pad(edge_index[0].astype(jnp.int32).reshape(NW, ERW),
                   pad).reshape(-1)
    cols = jnp.pad(edge_index[1].astype(jnp.int32).reshape(NW, ERW),
                   pad).reshape(-1)
    ev = jnp.pad(edge_vals.astype(jnp.float32).reshape(NW, ERW),
                 pad).reshape(-1)
    rs_part = _rsum_call(rows, ev)
    we, rc = _we_call(rs_part, rows, cols, ev)
    we3 = we.reshape(NW, NCH, CH)
    rc3 = jnp.pad(rc.reshape(NW, NCH, CH), ((0, 0), (0, NB), (0, 0)))

    Xf = X.astype(jnp.float32)
    xp = _pack_bf16(Xf)
    H = None
    b2 = bias.reshape(1, F).astype(jnp.float32)
    for k in range(3):
        axp = _hop_call(xp, rc3, we3)
        wk = weights[k].astype(jnp.float32)
        if k == 0:
            xp, H = _tc_first(axp[0], axp[1], wk)
        elif k == 1:
            xp, H = _tc_mid(axp[0], axp[1], wk, H)
        else:
            xp, H = _tc_last(axp[0], axp[1], wk, H, b2)
    return H


# speedup vs baseline: 1.2332x; 1.2332x over previous
import functools

import jax
import jax.numpy as jnp
from jax import lax
from jax.experimental import pallas as pl
from jax.experimental.pallas import tpu as pltpu
from jax.experimental.pallas import tpu_sc as plsc

N = 10000
E = 320000
F = 128
FW = F // 2
NC = 2
NS = 16
NW = NC * NS
ERW = E // NW
CH = 128
NB = 2
EPW = 10240
NCH = EPW // CH
HCH = NCH // 2
NG = HCH // NB
L = 16
NPAD = 10240
NPS = NPAD // NS
ZR = 16

_MESH = dict(core_axis_name="c", subcore_axis_name="s", num_cores=NC,
             num_subcores=NS)


def _zero_1d(ref, n):
    def body(i, _):
        ref[pl.ds(i * L, L)] = jnp.zeros((L,), jnp.float32)
        return 0
    lax.fori_loop(0, n // L, body, 0)


def _rsum_body(rows_hbm, vals_hbm, out_hbm, rows_v, vals_v, acc_v, part_v,
               red_v, shared):
    c = lax.axis_index("c")
    s = lax.axis_index("s")
    wid = c * NS + s
    _zero_1d(acc_v, NPAD)
    pltpu.sync_copy(rows_hbm.at[pl.ds(wid * EPW, EPW)], rows_v)
    pltpu.sync_copy(vals_hbm.at[pl.ds(wid * EPW, EPW)], vals_v)

    def body(i, _):
        idx = rows_v[pl.ds(i * L, L)]
        v = vals_v[pl.ds(i * L, L)]
        plsc.addupdate_scatter(acc_v, [idx], v)
        return 0
    lax.fori_loop(0, EPW // L, body, 0)

    pltpu.sync_copy(acc_v, shared.at[s])
    plsc.subcore_barrier()
    _zero_1d(red_v, NPS)
    for t in range(NS):
        pltpu.sync_copy(shared.at[t, pl.ds(s * NPS, NPS)], part_v)

        def addb(i, _):
            red_v[pl.ds(i * L, L)] = (red_v[pl.ds(i * L, L)]
                                      + part_v[pl.ds(i * L, L)])
            return 0
        lax.fori_loop(0, NPS // L, addb, 0)
    pltpu.sync_copy(red_v, out_hbm.at[c, pl.ds(s * NPS, NPS)])


_rsum_call = functools.partial(
    pl.kernel,
    out_type=jax.ShapeDtypeStruct((NC, NPAD), jnp.float32),
    mesh=plsc.VectorSubcoreMesh(**_MESH),
    compiler_params=pltpu.CompilerParams(needs_layout_passes=False),
    scratch_types=[
        pltpu.VMEM((EPW,), jnp.int32),
        pltpu.VMEM((EPW,), jnp.float32),
        pltpu.VMEM((NPAD,), jnp.float32),
        pltpu.VMEM((NPS,), jnp.float32),
        pltpu.VMEM((NPS,), jnp.float32),
        pltpu.VMEM_SHARED((NS, NPAD), jnp.float32),
    ],
)(_rsum_body)


def _we_body(rs_hbm, rows_hbm, cols_hbm, vals_hbm, we_hbm, rc_hbm, rs0_v,
             rs1_v, d_v, rows_v, cols_v, vals_v, we_v, rc_v):
    c = lax.axis_index("c")
    s = lax.axis_index("s")
    wid = c * NS + s
    pltpu.sync_copy(rs_hbm.at[0], rs0_v)
    pltpu.sync_copy(rs_hbm.at[1], rs1_v)

    def dbody(i, _):
        x = rs0_v[pl.ds(i * L, L)] + rs1_v[pl.ds(i * L, L)] + 1.0
        xi = plsc.bitcast(x, jnp.int32)
        yi = 0x5F3759DF - lax.shift_right_arithmetic(xi, 1)
        y = plsc.bitcast(yi, jnp.float32)
        hx = 0.5 * x
        for _ in range(4):
            y = y * (1.5 - hx * y * y)
        d_v[pl.ds(i * L, L)] = y
        return 0
    lax.fori_loop(0, NPAD // L, dbody, 0)

    pltpu.sync_copy(rows_hbm.at[pl.ds(wid * EPW, EPW)], rows_v)
    pltpu.sync_copy(cols_hbm.at[pl.ds(wid * EPW, EPW)], cols_v)
    pltpu.sync_copy(vals_hbm.at[pl.ds(wid * EPW, EPW)], vals_v)

    def ebody(i, _):
        r = rows_v[pl.ds(i * L, L)]
        cc = cols_v[pl.ds(i * L, L)]
        dr = plsc.load_gather(d_v, [r])
        dc = plsc.load_gather(d_v, [cc])
        we_v[pl.ds(i * L, L)] = vals_v[pl.ds(i * L, L)] * dr * dc
        rc_v[pl.ds(i * L, L)] = lax.bitwise_or(
            r, lax.shift_left(cc, jnp.int32(16)))
        return 0
    lax.fori_loop(0, EPW // L, ebody, 0)
    pltpu.sync_copy(we_v, we_hbm.at[pl.ds(wid * EPW, EPW)])
    pltpu.sync_copy(rc_v, rc_hbm.at[pl.ds(wid * EPW, EPW)])


_we_call = functools.partial(
    pl.kernel,
    out_type=[jax.ShapeDtypeStruct((NW * EPW,), jnp.float32),
              jax.ShapeDtypeStruct((NW * EPW,), jnp.int32)],
    mesh=plsc.VectorSubcoreMesh(**_MESH),
    compiler_params=pltpu.CompilerParams(needs_layout_passes=False),
    scratch_types=[
        pltpu.VMEM((NPAD,), jnp.float32),
        pltpu.VMEM((NPAD,), jnp.float32),
        pltpu.VMEM((NPAD,), jnp.float32),
        pltpu.VMEM((EPW,), jnp.int32),
        pltpu.VMEM((EPW,), jnp.int32),
        pltpu.VMEM((EPW,), jnp.float32),
        pltpu.VMEM((EPW,), jnp.float32),
        pltpu.VMEM((EPW,), jnp.int32),
    ],
)(_we_body)


def _hop_body(x_hbm, rc3_hbm, we3_hbm, out_hbm, rc2_v, we2_v, cb0, cb1,
              rb0, rb1, gb0, gb1, sbuf_v, zbuf_v, acc_sh, sg0, sg1, ssem):
    c = lax.axis_index("c")
    s = lax.axis_index("s")
    wid = c * NS + s
    gbufs = (gb0, gb1)
    colbs = (cb0, cb1)
    rowbs = (rb0, rb1)
    gsems = (sg0, sg1)

    def zrow(r, _):
        for g in range(F // L):
            zbuf_v[r, pl.ds(g * L, L)] = jnp.zeros((L,), jnp.float32)
        return 0
    lax.fori_loop(0, ZR, zrow, 0)
    for b in range(NPS // ZR):
        pltpu.sync_copy(zbuf_v, acc_sh.at[pl.ds(s * NPS + b * ZR, ZR)])
    plsc.subcore_barrier()

    def unpack_cols(j, cb):
        def ub(q, _):
            v = rc2_v[j, pl.ds(q * L, L)]
            cb[pl.ds(q * L, L)] = lax.shift_right_logical(v, jnp.int32(16))
            return 0
        lax.fori_loop(0, CH // L, ub, 0)

    def unpack_rows(j, rb):
        def ub(q, _):
            v = rc2_v[j, pl.ds(q * L, L)]
            rb[pl.ds(q * L, L)] = lax.bitwise_and(v, jnp.int32(0xFFFF))
            return 0
        lax.fori_loop(0, CH // L, ub, 0)

    def mult(src, j):
        def qbody(q, _):
            wv = we2_v[j, pl.ds(q * L, L)]
            for lane in range(L):
                w = wv[lane]
                e = q * L + lane
                for g in range(F // 32):
                    v = src[e, pl.ds(g * L, L)]
                    lo = plsc.bitcast(
                        lax.shift_left(v, jnp.int32(16)), jnp.float32)
                    hi = plsc.bitcast(
                        lax.bitwise_and(v, jnp.int32(-65536)), jnp.float32)
                    sbuf_v[e, pl.ds(g * 32, L)] = lo * w
                    sbuf_v[e, pl.ds(g * 32 + L, L)] = hi * w
            return 0
        lax.fori_loop(0, CH // L, qbody, 0)

    def fire(j, b):
        unpack_cols(j, colbs[b])
        pltpu.async_copy(x_hbm.at[colbs[b]], gbufs[b], gsems[b])

    def consume(j, b, first):
        pltpu.make_async_copy(x_hbm.at[colbs[b]], gbufs[b], gsems[b]).wait()
        if not first:
            pltpu.make_async_copy(sbuf_v, acc_sh.at[rowbs[b]], ssem).wait()
        unpack_rows(j, rowbs[b])
        mult(gbufs[b], j)
        pltpu.async_copy(sbuf_v, acc_sh.at[rowbs[b]], ssem, add=True)

    for p in range(2):
        pltpu.sync_copy(rc3_hbm.at[wid, pl.ds(p * HCH, HCH)], rc2_v)
        pltpu.sync_copy(we3_hbm.at[wid, pl.ds(p * HCH, HCH)], we2_v)
        for b in range(NB):
            fire(b, b)
        for b in range(NB):
            consume(b, b, p == 0 and b == 0)

        def group(g, _):
            for b in range(NB):
                fire(g * NB + b, b)
            for b in range(NB):
                consume(g * NB + b, b, False)
            return 0
        lax.fori_loop(1, NG, group, 0)

    pltpu.make_async_copy(sbuf_v, acc_sh.at[rowbs[1]], ssem).wait()
    plsc.subcore_barrier()
    pltpu.sync_copy(acc_sh.at[pl.ds(s * NPS, NPS)],
                    out_hbm.at[c, pl.ds(s * NPS, NPS)])


_hop_call = functools.partial(
    pl.kernel,
    out_type=jax.ShapeDtypeStruct((NC, NPAD, F), jnp.float32),
    mesh=plsc.VectorSubcoreMesh(**_MESH),
    compiler_params=pltpu.CompilerParams(needs_layout_passes=False,
                                         use_tc_tiling_on_sc=False),
    scratch_types=[
        pltpu.VMEM((HCH, CH), jnp.int32),
        pltpu.VMEM((HCH, CH), jnp.float32),
        pltpu.VMEM((CH,), jnp.int32),
        pltpu.VMEM((CH,), jnp.int32),
        pltpu.VMEM((CH,), jnp.int32),
        pltpu.VMEM((CH,), jnp.int32),
        pltpu.VMEM((CH, FW), jnp.int32),
        pltpu.VMEM((CH, FW), jnp.int32),
        pltpu.VMEM((CH, F), jnp.float32),
        pltpu.VMEM((ZR, F), jnp.float32),
        pltpu.VMEM_SHARED((NPAD, F), jnp.float32),
        pltpu.SemaphoreType.DMA,
        pltpu.SemaphoreType.DMA,
        pltpu.SemaphoreType.DMA,
    ],
)(_hop_body)


_RB = 1000


def _pack_bf16(x):
    u = lax.bitcast_convert_type(x.astype(jnp.bfloat16), jnp.uint16)
    u = u.astype(jnp.uint32)
    parts = []
    for g in range(F // 32):
        lo = u[:, g * 32:g * 32 + 16]
        hi = u[:, g * 32 + 16:g * 32 + 32]
        parts.append(lax.bitwise_or(lax.shift_left(hi, jnp.uint32(16)), lo))
    return lax.bitcast_convert_type(jnp.concatenate(parts, axis=1),
                                    jnp.int32)


def _mk_tc(first, last):
    def body(*refs):
        a0, a1, w = refs[:3]
        rest = list(refs[3:])
        h_prev = None if first else rest.pop(0)[...]
        b = rest.pop(0)[...] if last else None
        xp_o, h_o = rest
        xa = a0[...] + a1[...]
        xp_o[...] = _pack_bf16(xa)
        acc = jnp.dot(xa, w[...], preferred_element_type=jnp.float32)
        if h_prev is not None:
            acc = acc + h_prev
        if b is not None:
            acc = acc + b
        h_o[...] = acc

    row_spec = pl.BlockSpec((_RB, F), lambda i: (i, 0))
    pack_spec = pl.BlockSpec((_RB, FW), lambda i: (i, 0))
    w_spec = pl.BlockSpec((F, F), lambda i: (0, 0))
    bias_spec = pl.BlockSpec((1, F), lambda i: (0, 0))
    in_specs = [row_spec, row_spec, w_spec]
    if not first:
        in_specs.append(row_spec)
    if last:
        in_specs.append(bias_spec)
    return pl.pallas_call(
        body,
        grid=(N // _RB,),
        in_specs=in_specs,
        out_specs=[pack_spec, row_spec],
        out_shape=[jax.ShapeDtypeStruct((N, FW), jnp.int32),
                   jax.ShapeDtypeStruct((N, F), jnp.float32)],
    )


_tc_first = _mk_tc(True, False)
_tc_mid = _mk_tc(False, False)
_tc_last = _mk_tc(False, True)


def kernel(edge_index, edge_vals, X, weights, bias):
    pad = ((0, 0), (0, EPW - ERW))
    rows = jnp.pad(edge_index[0].astype(jnp.int32).reshape(NW, ERW),
                   pad).reshape(-1)
    cols = jnp.pad(edge_index[1].astype(jnp.int32).reshape(NW, ERW),
                   pad).reshape(-1)
    ev = jnp.pad(edge_vals.astype(jnp.float32).reshape(NW, ERW),
                 pad).reshape(-1)
    rs_part = _rsum_call(rows, ev)
    we, rc = _we_call(rs_part, rows, cols, ev)
    we3 = we.reshape(NW, NCH, CH)
    rc3 = rc.reshape(NW, NCH, CH)

    Xf = X.astype(jnp.float32)
    xp = _pack_bf16(Xf)
    H = None
    b2 = bias.reshape(1, F).astype(jnp.float32)
    for k in range(3):
        axp = _hop_call(xp, rc3, we3)
        wk = weights[k].astype(jnp.float32)
        if k == 0:
            xp, H = _tc_first(axp[0], axp[1], wk)
        elif k == 1:
            xp, H = _tc_mid(axp[0], axp[1], wk, H)
        else:
            xp, H = _tc_last(axp[0], axp[1], wk, H, b2)
    return H


# speedup vs baseline: 1.2524x; 1.0155x over previous
import functools

import jax
import jax.numpy as jnp
from jax import lax
from jax.experimental import pallas as pl
from jax.experimental.pallas import tpu as pltpu
from jax.experimental.pallas import tpu_sc as plsc

N = 10000
E = 320000
F = 128
FW = F // 2
NC = 2
NS = 16
NW = NC * NS
ERW = E // NW
CH = 64
NB = 2
EPW = 10240
NCH = EPW // CH
HCH = NCH // 4
NG = HCH // NB
L = 16
NPAD = 10240
NPS = NPAD // NS
ZR = 16

_MESH = dict(core_axis_name="c", subcore_axis_name="s", num_cores=NC,
             num_subcores=NS)


def _zero_1d(ref, n):
    def body(i, _):
        ref[pl.ds(i * L, L)] = jnp.zeros((L,), jnp.float32)
        return 0
    lax.fori_loop(0, n // L, body, 0)


def _rsum_body(rows_hbm, vals_hbm, out_hbm, rows_v, vals_v, acc_v, part_v,
               red_v, shared):
    c = lax.axis_index("c")
    s = lax.axis_index("s")
    wid = c * NS + s
    _zero_1d(acc_v, NPAD)
    pltpu.sync_copy(rows_hbm.at[pl.ds(wid * EPW, EPW)], rows_v)
    pltpu.sync_copy(vals_hbm.at[pl.ds(wid * EPW, EPW)], vals_v)

    def body(i, _):
        idx = rows_v[pl.ds(i * L, L)]
        v = vals_v[pl.ds(i * L, L)]
        plsc.addupdate_scatter(acc_v, [idx], v)
        return 0
    lax.fori_loop(0, EPW // L, body, 0)

    pltpu.sync_copy(acc_v, shared.at[s])
    plsc.subcore_barrier()
    _zero_1d(red_v, NPS)
    for t in range(NS):
        pltpu.sync_copy(shared.at[t, pl.ds(s * NPS, NPS)], part_v)

        def addb(i, _):
            red_v[pl.ds(i * L, L)] = (red_v[pl.ds(i * L, L)]
                                      + part_v[pl.ds(i * L, L)])
            return 0
        lax.fori_loop(0, NPS // L, addb, 0)
    pltpu.sync_copy(red_v, out_hbm.at[c, pl.ds(s * NPS, NPS)])


_rsum_call = functools.partial(
    pl.kernel,
    out_type=jax.ShapeDtypeStruct((NC, NPAD), jnp.float32),
    mesh=plsc.VectorSubcoreMesh(**_MESH),
    compiler_params=pltpu.CompilerParams(needs_layout_passes=False),
    scratch_types=[
        pltpu.VMEM((EPW,), jnp.int32),
        pltpu.VMEM((EPW,), jnp.float32),
        pltpu.VMEM((NPAD,), jnp.float32),
        pltpu.VMEM((NPS,), jnp.float32),
        pltpu.VMEM((NPS,), jnp.float32),
        pltpu.VMEM_SHARED((NS, NPAD), jnp.float32),
    ],
)(_rsum_body)


def _we_body(rs_hbm, rows_hbm, cols_hbm, vals_hbm, we_hbm, rc_hbm, rs0_v,
             rs1_v, d_v, rows_v, cols_v, vals_v, we_v, rc_v):
    c = lax.axis_index("c")
    s = lax.axis_index("s")
    wid = c * NS + s
    pltpu.sync_copy(rs_hbm.at[0], rs0_v)
    pltpu.sync_copy(rs_hbm.at[1], rs1_v)

    def dbody(i, _):
        x = rs0_v[pl.ds(i * L, L)] + rs1_v[pl.ds(i * L, L)] + 1.0
        xi = plsc.bitcast(x, jnp.int32)
        yi = 0x5F3759DF - lax.shift_right_arithmetic(xi, 1)
        y = plsc.bitcast(yi, jnp.float32)
        hx = 0.5 * x
        for _ in range(4):
            y = y * (1.5 - hx * y * y)
        d_v[pl.ds(i * L, L)] = y
        return 0
    lax.fori_loop(0, NPAD // L, dbody, 0)

    pltpu.sync_copy(rows_hbm.at[pl.ds(wid * EPW, EPW)], rows_v)
    pltpu.sync_copy(cols_hbm.at[pl.ds(wid * EPW, EPW)], cols_v)
    pltpu.sync_copy(vals_hbm.at[pl.ds(wid * EPW, EPW)], vals_v)

    def ebody(i, _):
        r = rows_v[pl.ds(i * L, L)]
        cc = cols_v[pl.ds(i * L, L)]
        dr = plsc.load_gather(d_v, [r])
        dc = plsc.load_gather(d_v, [cc])
        we_v[pl.ds(i * L, L)] = vals_v[pl.ds(i * L, L)] * dr * dc
        rc_v[pl.ds(i * L, L)] = lax.bitwise_or(
            r, lax.shift_left(cc, jnp.int32(16)))
        return 0
    lax.fori_loop(0, EPW // L, ebody, 0)
    pltpu.sync_copy(we_v, we_hbm.at[pl.ds(wid * EPW, EPW)])
    pltpu.sync_copy(rc_v, rc_hbm.at[pl.ds(wid * EPW, EPW)])


_we_call = functools.partial(
    pl.kernel,
    out_type=[jax.ShapeDtypeStruct((NW * EPW,), jnp.float32),
              jax.ShapeDtypeStruct((NW * EPW,), jnp.int32)],
    mesh=plsc.VectorSubcoreMesh(**_MESH),
    compiler_params=pltpu.CompilerParams(needs_layout_passes=False),
    scratch_types=[
        pltpu.VMEM((NPAD,), jnp.float32),
        pltpu.VMEM((NPAD,), jnp.float32),
        pltpu.VMEM((NPAD,), jnp.float32),
        pltpu.VMEM((EPW,), jnp.int32),
        pltpu.VMEM((EPW,), jnp.int32),
        pltpu.VMEM((EPW,), jnp.float32),
        pltpu.VMEM((EPW,), jnp.float32),
        pltpu.VMEM((EPW,), jnp.int32),
    ],
)(_we_body)


def _hop_body(x_hbm, rc3_hbm, we3_hbm, out_hbm, rc2_v, we2_v, cb0, cb1,
              rb0, rb1, gb0, gb1, sb0, sb1, zbuf_v, acc_sh, sg0, sg1, ss0,
              ss1):
    c = lax.axis_index("c")
    s = lax.axis_index("s")
    wid = c * NS + s
    gbufs = (gb0, gb1)
    sbufs = (sb0, sb1)
    colbs = (cb0, cb1)
    rowbs = (rb0, rb1)
    gsems = (sg0, sg1)
    ssems = (ss0, ss1)

    def zrow(r, _):
        for g in range(F // L):
            zbuf_v[r, pl.ds(g * L, L)] = jnp.zeros((L,), jnp.float32)
        return 0
    lax.fori_loop(0, ZR, zrow, 0)
    for b in range(NPS // ZR):
        pltpu.sync_copy(zbuf_v, acc_sh.at[pl.ds(s * NPS + b * ZR, ZR)])
    plsc.subcore_barrier()

    def unpack_cols(j, cb):
        def ub(q, _):
            v = rc2_v[j, pl.ds(q * L, L)]
            cb[pl.ds(q * L, L)] = lax.shift_right_logical(v, jnp.int32(16))
            return 0
        lax.fori_loop(0, CH // L, ub, 0)

    def unpack_rows(j, rb):
        def ub(q, _):
            v = rc2_v[j, pl.ds(q * L, L)]
            rb[pl.ds(q * L, L)] = lax.bitwise_and(v, jnp.int32(0xFFFF))
            return 0
        lax.fori_loop(0, CH // L, ub, 0)

    def mult(src, dst, j):
        def qbody(q, _):
            wv = we2_v[j, pl.ds(q * L, L)]
            for lane in range(L):
                w = wv[lane]
                e = q * L + lane
                for g in range(F // 32):
                    v = src[e, pl.ds(g * L, L)]
                    lo = plsc.bitcast(
                        lax.shift_left(v, jnp.int32(16)), jnp.float32)
                    hi = plsc.bitcast(
                        lax.bitwise_and(v, jnp.int32(-65536)), jnp.float32)
                    dst[e, pl.ds(g * 32, L)] = lo * w
                    dst[e, pl.ds(g * 32 + L, L)] = hi * w
            return 0
        lax.fori_loop(0, CH // L, qbody, 0)

    def fire(j, b):
        unpack_cols(j, colbs[b])
        pltpu.async_copy(x_hbm.at[colbs[b]], gbufs[b], gsems[b])

    def consume(j, b, first):
        pltpu.make_async_copy(x_hbm.at[colbs[b]], gbufs[b], gsems[b]).wait()
        if not first:
            pltpu.make_async_copy(sbufs[b], acc_sh.at[rowbs[b]],
                                  ssems[b]).wait()
        unpack_rows(j, rowbs[b])
        mult(gbufs[b], sbufs[b], j)
        pltpu.async_copy(sbufs[b], acc_sh.at[rowbs[b]], ssems[b], add=True)

    for p in range(NCH // HCH):
        pltpu.sync_copy(rc3_hbm.at[wid, pl.ds(p * HCH, HCH)], rc2_v)
        pltpu.sync_copy(we3_hbm.at[wid, pl.ds(p * HCH, HCH)], we2_v)
        for b in range(NB):
            fire(b, b)
        for b in range(NB):
            consume(b, b, p == 0)

        def group(g, _):
            for b in range(NB):
                fire(g * NB + b, b)
            for b in range(NB):
                consume(g * NB + b, b, False)
            return 0
        lax.fori_loop(1, NG, group, 0)

    for b in range(NB):
        pltpu.make_async_copy(sbufs[b], acc_sh.at[rowbs[b]], ssems[b]).wait()
    plsc.subcore_barrier()
    pltpu.sync_copy(acc_sh.at[pl.ds(s * NPS, NPS)],
                    out_hbm.at[c, pl.ds(s * NPS, NPS)])


_hop_call = functools.partial(
    pl.kernel,
    out_type=jax.ShapeDtypeStruct((NC, NPAD, F), jnp.float32),
    mesh=plsc.VectorSubcoreMesh(**_MESH),
    compiler_params=pltpu.CompilerParams(needs_layout_passes=False,
                                         use_tc_tiling_on_sc=False),
    scratch_types=[
        pltpu.VMEM((HCH, CH), jnp.int32),
        pltpu.VMEM((HCH, CH), jnp.float32),
        pltpu.VMEM((CH,), jnp.int32),
        pltpu.VMEM((CH,), jnp.int32),
        pltpu.VMEM((CH,), jnp.int32),
        pltpu.VMEM((CH,), jnp.int32),
        pltpu.VMEM((CH, FW), jnp.int32),
        pltpu.VMEM((CH, FW), jnp.int32),
        pltpu.VMEM((CH, F), jnp.float32),
        pltpu.VMEM((CH, F), jnp.float32),
        pltpu.VMEM((ZR, F), jnp.float32),
        pltpu.VMEM_SHARED((NPAD, F), jnp.float32),
        pltpu.SemaphoreType.DMA,
        pltpu.SemaphoreType.DMA,
        pltpu.SemaphoreType.DMA,
        pltpu.SemaphoreType.DMA,
    ],
)(_hop_body)


_RB = 1000


def _pack_bf16(x):
    u = lax.bitcast_convert_type(x.astype(jnp.bfloat16), jnp.uint16)
    u = u.astype(jnp.uint32)
    parts = []
    for g in range(F // 32):
        lo = u[:, g * 32:g * 32 + 16]
        hi = u[:, g * 32 + 16:g * 32 + 32]
        parts.append(lax.bitwise_or(lax.shift_left(hi, jnp.uint32(16)), lo))
    return lax.bitcast_convert_type(jnp.concatenate(parts, axis=1),
                                    jnp.int32)


def _mk_tc(first, last):
    def body(*refs):
        a0, a1, w = refs[:3]
        rest = list(refs[3:])
        h_prev = None if first else rest.pop(0)[...]
        b = rest.pop(0)[...] if last else None
        xp_o, h_o = rest
        xa = a0[...] + a1[...]
        xp_o[...] = _pack_bf16(xa)
        acc = jnp.dot(xa, w[...], preferred_element_type=jnp.float32)
        if h_prev is not None:
            acc = acc + h_prev
        if b is not None:
            acc = acc + b
        h_o[...] = acc

    row_spec = pl.BlockSpec((_RB, F), lambda i: (i, 0))
    pack_spec = pl.BlockSpec((_RB, FW), lambda i: (i, 0))
    w_spec = pl.BlockSpec((F, F), lambda i: (0, 0))
    bias_spec = pl.BlockSpec((1, F), lambda i: (0, 0))
    in_specs = [row_spec, row_spec, w_spec]
    if not first:
        in_specs.append(row_spec)
    if last:
        in_specs.append(bias_spec)
    return pl.pallas_call(
        body,
        grid=(N // _RB,),
        in_specs=in_specs,
        out_specs=[pack_spec, row_spec],
        out_shape=[jax.ShapeDtypeStruct((N, FW), jnp.int32),
                   jax.ShapeDtypeStruct((N, F), jnp.float32)],
    )


_tc_first = _mk_tc(True, False)
_tc_mid = _mk_tc(False, False)
_tc_last = _mk_tc(False, True)


def kernel(edge_index, edge_vals, X, weights, bias):
    pad = ((0, 0), (0, EPW - ERW))
    rows = jnp.pad(edge_index[0].astype(jnp.int32).reshape(NW, ERW),
                   pad).reshape(-1)
    cols = jnp.pad(edge_index[1].astype(jnp.int32).reshape(NW, ERW),
                   pad).reshape(-1)
    ev = jnp.pad(edge_vals.astype(jnp.float32).reshape(NW, ERW),
                 pad).reshape(-1)
    rs_part = _rsum_call(rows, ev)
    we, rc = _we_call(rs_part, rows, cols, ev)
    we3 = we.reshape(NW, NCH, CH)
    rc3 = rc.reshape(NW, NCH, CH)

    Xf = X.astype(jnp.float32)
    xp = _pack_bf16(Xf)
    H = None
    b2 = bias.reshape(1, F).astype(jnp.float32)
    for k in range(3):
        axp = _hop_call(xp, rc3, we3)
        wk = weights[k].astype(jnp.float32)
        if k == 0:
            xp, H = _tc_first(axp[0], axp[1], wk)
        elif k == 1:
            xp, H = _tc_mid(axp[0], axp[1], wk, H)
        else:
            xp, H = _tc_last(axp[0], axp[1], wk, H, b2)
    return H
